# baseline (speedup 1.0000x reference)
import functools

import jax
import jax.numpy as jnp
from jax.experimental import pallas as pl
from jax.experimental.pallas import tpu as pltpu
from jax.experimental.pallas import tpu_sc as plsc

_SC_CHUNK = 16


def _sc_partial(x2d):
    n_rows, h = x2d.shape
    n_chunks = n_rows // _SC_CHUNK
    mesh = plsc.VectorSubcoreMesh(core_axis_name="core",
                                  subcore_axis_name="subcore")

    @functools.partial(
        pl.kernel,
        out_type=jax.ShapeDtypeStruct((n_chunks, h), jnp.float32),
        mesh=mesh)
    def sc_kernel(x_hbm, o_hbm):
        def body(x_vmem, o_vmem):
            @pl.loop(0, h, step=16)
            def _(c):
                acc = x_vmem.at[0, pl.ds(c, 16)][...]
                for r in range(1, _SC_CHUNK):
                    acc = acc + x_vmem.at[r, pl.ds(c, 16)][...]
                o_vmem.at[0, pl.ds(c, 16)][...] = acc

        pltpu.emit_pipeline(
            body,
            grid=(n_chunks,),
            in_specs=[pl.BlockSpec((_SC_CHUNK, h), lambda i: (i, 0))],
            out_specs=[pl.BlockSpec((1, h), lambda i: (i, 0))],
            core_axis_name=("core", "subcore"),
            dimension_semantics=(pltpu.PARALLEL,),
        )(x_hbm, o_hbm)

    return sc_kernel(x2d)


@jax.jit
def kernel(hidden_states, W1, b1, W2, b2):
    B, S, H = hidden_states.shape
    x2d = hidden_states.reshape(B * S, H)
    part = _sc_partial(x2d)
    fv = part.reshape(B, S // _SC_CHUNK, H).sum(axis=1) / S
    h = jnp.maximum(fv @ W1 + b1, 0.0)
    logits = h @ W2 + b2
    rw = jax.nn.softmax(logits, axis=-1)
    _, idx = jax.lax.top_k(rw, 2)
    return rw, idx


# speedup vs baseline: 1.4917x; 1.4917x over previous
import functools

import jax
import jax.numpy as jnp
from jax.experimental import pallas as pl
from jax.experimental.pallas import tpu as pltpu
from jax.experimental.pallas import tpu_sc as plsc

_SC_BATCHES = 1
_SC_CHUNK = 16
_TC_CH = 256
_TC_STREAMS = 4
_W1_CH = 256


def _sc_partial(x2d, split):
    n_rows, h = x2d.shape
    n_chunks = (n_rows - split) // _SC_CHUNK
    off = split // _SC_CHUNK
    mesh = plsc.VectorSubcoreMesh(core_axis_name="core",
                                  subcore_axis_name="subcore")

    @functools.partial(
        pl.kernel,
        out_type=jax.ShapeDtypeStruct((n_chunks, h), jnp.float32),
        mesh=mesh)
    def sc_kernel(x_hbm, o_hbm):
        def body(x_vmem, o_vmem):
            @pl.loop(0, h, step=16)
            def _(c):
                acc = x_vmem.at[0, pl.ds(c, 16)][...]
                for r in range(1, _SC_CHUNK):
                    acc = acc + x_vmem.at[r, pl.ds(c, 16)][...]
                o_vmem.at[0, pl.ds(c, 16)][...] = acc

        pltpu.emit_pipeline(
            body,
            grid=(n_chunks,),
            in_specs=[pl.BlockSpec((_SC_CHUNK, h), lambda i: (i + off, 0))],
            out_specs=[pl.BlockSpec((1, h), lambda i: (i, 0))],
            core_axis_name=("core", "subcore"),
            dimension_semantics=(pltpu.PARALLEL,),
        )(x_hbm, o_hbm)

    return sc_kernel(x2d)


def _tc_reduce_body(*refs, n_steps):
    xs = refs[:_TC_STREAMS]
    acc_ref = refs[_TC_STREAMS]
    i = pl.program_id(0)
    ones = jnp.ones((1, _TC_CH), jnp.float32)
    for s, x_ref in enumerate(xs):
        acc_ref[pl.ds(i + s * n_steps, 1), :] = jnp.dot(
            ones, x_ref[...], preferred_element_type=jnp.float32)


def _tc_reduce(x2d, split):
    h = x2d.shape[1]
    n_chunks = split // _TC_CH
    n_steps = n_chunks // _TC_STREAMS
    body = functools.partial(_tc_reduce_body, n_steps=n_steps)
    return pl.pallas_call(
        body,
        grid=(n_steps,),
        in_specs=[
            pl.BlockSpec((_TC_CH, h), functools.partial(
                lambda s, i: (i + s * n_steps, 0), s))
            for s in range(_TC_STREAMS)
        ],
        out_specs=pl.BlockSpec((n_chunks, h), lambda i: (0, 0)),
        out_shape=jax.ShapeDtypeStruct((n_chunks, h), jnp.float32),
    )(*([x2d] * _TC_STREAMS))


def _tc_mlp_body(head_ref, tail_ref, w1_hbm, b1_ref, w2_ref, b2_ref,
                 rw_ref, idx_ref, w1_vmem, w1_sems,
                 *, n_w1, b, n_sc_b, s_total):
    def w1_copy(j):
        return pltpu.make_async_copy(
            w1_hbm.at[pl.ds(j * _W1_CH, _W1_CH), :],
            w1_vmem.at[pl.ds(j * _W1_CH, _W1_CH), :],
            w1_sems.at[j])

    for j in range(n_w1):
        w1_copy(j).start()

    nh = head_ref.shape[0]
    nt = tail_ref.shape[0]
    h_per_b = nh // (b - n_sc_b)
    t_per_b = nt // n_sc_b
    inv_s = 1.0 / s_total

    rows = jax.lax.broadcasted_iota(jnp.int32, (b, nh), 0)
    cols = jax.lax.broadcasted_iota(jnp.int32, (b, nh), 1)
    sel_h = (cols // h_per_b == rows).astype(jnp.float32) * inv_s
    rows_t = jax.lax.broadcasted_iota(jnp.int32, (b, nt), 0)
    cols_t = jax.lax.broadcasted_iota(jnp.int32, (b, nt), 1)
    sel_t = (cols_t // t_per_b + (b - n_sc_b) == rows_t)
    sel_t = sel_t.astype(jnp.float32) * inv_s

    fv = (jnp.dot(sel_h, head_ref[...], preferred_element_type=jnp.float32)
          + jnp.dot(sel_t, tail_ref[...], preferred_element_type=jnp.float32))

    for j in range(n_w1):
        w1_copy(j).wait()

    h = jnp.dot(fv, w1_vmem[...], preferred_element_type=jnp.float32)
    h = jnp.maximum(h + b1_ref[...], 0.0)
    logits = jnp.dot(h, w2_ref[...], preferred_element_type=jnp.float32)
    logits = logits + b2_ref[...]
    m = jnp.max(logits, axis=-1, keepdims=True)
    e = jnp.exp(logits - m)
    w = e / jnp.sum(e, axis=-1, keepdims=True)
    rw_ref[...] = w
    ncols = w.shape[-1]
    ids = jax.lax.broadcasted_iota(jnp.int32, w.shape, 1)
    m1 = jnp.max(w, axis=-1, keepdims=True)
    i1 = jnp.min(jnp.where(w == m1, ids, ncols), axis=-1, keepdims=True)
    wm = jnp.where(ids == i1, -jnp.inf, w)
    m2 = jnp.max(wm, axis=-1, keepdims=True)
    i2 = jnp.min(jnp.where(wm == m2, ids, ncols), axis=-1, keepdims=True)
    col = jax.lax.broadcasted_iota(jnp.int32, idx_ref.shape, 1)
    idx_ref[...] = jnp.where(col == 0, i1, i2)


@jax.jit
def kernel(hidden_states, W1, b1, W2, b2):
    B, S, H = hidden_states.shape
    E = W2.shape[1]
    x2d = hidden_states.reshape(B * S, H)
    split = (B - _SC_BATCHES) * S
    n_w1 = H // _W1_CH

    sc_part = _sc_partial(x2d, split)
    head_part = _tc_reduce(x2d, split)

    b1r = b1.reshape(1, H)
    b2r = b2.reshape(1, E)

    body = functools.partial(
        _tc_mlp_body, n_w1=n_w1, b=B, n_sc_b=_SC_BATCHES, s_total=S)

    rw, idx = pl.pallas_call(
        body,
        grid=(1,),
        in_specs=[
            pl.BlockSpec(head_part.shape, lambda i: (0, 0)),
            pl.BlockSpec(sc_part.shape, lambda i: (0, 0)),
            pl.BlockSpec(memory_space=pl.ANY),
            pl.BlockSpec((1, H), lambda i: (0, 0)),
            pl.BlockSpec((H, E), lambda i: (0, 0)),
            pl.BlockSpec((1, E), lambda i: (0, 0)),
        ],
        out_specs=[
            pl.BlockSpec((B, E), lambda i: (0, 0)),
            pl.BlockSpec((B, 2), lambda i: (0, 0)),
        ],
        out_shape=[
            jax.ShapeDtypeStruct((B, E), jnp.float32),
            jax.ShapeDtypeStruct((B, 2), jnp.int32),
        ],
        scratch_shapes=[
            pltpu.VMEM((H, H), jnp.float32),
            pltpu.SemaphoreType.DMA((n_w1,)),
        ],
    )(head_part, sc_part, W1, b1r, W2, b2r)
    return rw, idx


# speedup vs baseline: 2.2035x; 1.4772x over previous
import functools

import jax
import jax.numpy as jnp
from jax.experimental import pallas as pl
from jax.experimental.pallas import tpu as pltpu

_CH = 128
_N_BUF = 16
_W1_CH = 256


def _router_body(x_hbm, w1_hbm, b1_ref, w2_ref, b2_ref,
                 rw_ref, idx_ref,
                 acc_ref, bufs_ref, w1_vmem, x_sems, w1_sems,
                 *, n_chunks, n_w1, b, blk_per_b, s_total):
    def x_copy(c):
        return pltpu.make_async_copy(
            x_hbm.at[pl.ds(c * _CH, _CH), :],
            bufs_ref.at[c % _N_BUF],
            x_sems.at[c % _N_BUF])

    def w1_copy(j):
        return pltpu.make_async_copy(
            w1_hbm.at[pl.ds(j * _W1_CH, _W1_CH), :],
            w1_vmem.at[pl.ds(j * _W1_CH, _W1_CH), :],
            w1_sems.at[j])

    for c in range(_N_BUF):
        x_copy(c).start()
    for j in range(n_w1):
        w1_copy(j).start()

    ones = jnp.ones((1, _CH), jnp.float32)
    for c in range(n_chunks):
        x_copy(c).wait()
        acc_ref[pl.ds(c, 1), :] = jnp.dot(
            ones, bufs_ref[c % _N_BUF],
            preferred_element_type=jnp.float32)
        if c + _N_BUF < n_chunks:
            x_copy(c + _N_BUF).start()

    for j in range(n_w1):
        w1_copy(j).wait()

    nb = acc_ref.shape[0]
    rows = jax.lax.broadcasted_iota(jnp.int32, (b, nb), 0)
    cols = jax.lax.broadcasted_iota(jnp.int32, (b, nb), 1)
    sel = (cols // blk_per_b == rows).astype(jnp.float32) * (1.0 / s_total)
    fv = jnp.dot(sel, acc_ref[...], preferred_element_type=jnp.float32)
    h = jnp.dot(fv, w1_vmem[...], preferred_element_type=jnp.float32)
    h = jnp.maximum(h + b1_ref[...], 0.0)
    logits = jnp.dot(h, w2_ref[...], preferred_element_type=jnp.float32)
    logits = logits + b2_ref[...]
    m = jnp.max(logits, axis=-1, keepdims=True)
    e = jnp.exp(logits - m)
    w = e / jnp.sum(e, axis=-1, keepdims=True)
    rw_ref[...] = w
    ncols = w.shape[-1]
    ids = jax.lax.broadcasted_iota(jnp.int32, w.shape, 1)
    m1 = jnp.max(w, axis=-1, keepdims=True)
    i1 = jnp.min(jnp.where(w == m1, ids, ncols), axis=-1, keepdims=True)
    wm = jnp.where(ids == i1, -jnp.inf, w)
    m2 = jnp.max(wm, axis=-1, keepdims=True)
    i2 = jnp.min(jnp.where(wm == m2, ids, ncols), axis=-1, keepdims=True)
    col = jax.lax.broadcasted_iota(jnp.int32, idx_ref.shape, 1)
    idx_ref[...] = jnp.where(col == 0, i1, i2)


@jax.jit
def kernel(hidden_states, W1, b1, W2, b2):
    B, S, H = hidden_states.shape
    E = W2.shape[1]
    x2d = hidden_states.reshape(B * S, H)
    n_chunks = (B * S) // _CH
    n_w1 = H // _W1_CH
    blk_per_b = S // _CH

    b1r = b1.reshape(1, H)
    b2r = b2.reshape(1, E)

    body = functools.partial(
        _router_body, n_chunks=n_chunks, n_w1=n_w1, b=B,
        blk_per_b=blk_per_b, s_total=S)

    rw, idx = pl.pallas_call(
        body,
        grid=(1,),
        in_specs=[
            pl.BlockSpec(memory_space=pl.ANY),
            pl.BlockSpec(memory_space=pl.ANY),
            pl.BlockSpec((1, H), lambda i: (0, 0)),
            pl.BlockSpec((H, E), lambda i: (0, 0)),
            pl.BlockSpec((1, E), lambda i: (0, 0)),
        ],
        out_specs=[
            pl.BlockSpec((B, E), lambda i: (0, 0)),
            pl.BlockSpec((B, 2), lambda i: (0, 0)),
        ],
        out_shape=[
            jax.ShapeDtypeStruct((B, E), jnp.float32),
            jax.ShapeDtypeStruct((B, 2), jnp.int32),
        ],
        scratch_shapes=[
            pltpu.VMEM((n_chunks, H), jnp.float32),
            pltpu.VMEM((_N_BUF, _CH, H), jnp.float32),
            pltpu.VMEM((H, H), jnp.float32),
            pltpu.SemaphoreType.DMA((_N_BUF,)),
            pltpu.SemaphoreType.DMA((n_w1,)),
        ],
    )(x2d, W1, b1r, W2, b2r)
    return rw, idx


# speedup vs baseline: 2.3303x; 1.0575x over previous
import functools

import jax
import jax.numpy as jnp
from jax.experimental import pallas as pl
from jax.experimental.pallas import tpu as pltpu

_R_BLK = 512
_N_STREAMS = 2


def _router_body(*refs, n_steps, n_blk, blk_per_b, s_total):
    xs = refs[:_N_STREAMS]
    w1_hbm, b1_ref, w2_ref, b2_ref, rw_ref, idx_ref = refs[_N_STREAMS:-3]
    acc_ref, w1_vmem, dma_sem = refs[-3:]
    i = pl.program_id(0)

    @pl.when(i == 0)
    def _init():
        pltpu.make_async_copy(w1_hbm, w1_vmem, dma_sem).start()

    for s, x_ref in enumerate(xs):
        ones = jnp.ones((1, x_ref.shape[0]), jnp.float32)
        acc_ref[pl.ds(i + s * n_steps, 1), :] = jnp.dot(
            ones, x_ref[...], preferred_element_type=jnp.float32)

    @pl.when(i == n_steps - 1)
    def _finish():
        b = acc_ref.shape[0]
        nb = b // blk_per_b
        rows = jax.lax.broadcasted_iota(jnp.int32, (nb, b), 0)
        cols = jax.lax.broadcasted_iota(jnp.int32, (nb, b), 1)
        sel = (cols // blk_per_b == rows).astype(jnp.float32) * (1.0 / s_total)
        fv = jnp.dot(sel, acc_ref[...], preferred_element_type=jnp.float32)
        pltpu.make_async_copy(w1_hbm, w1_vmem, dma_sem).wait()
        h = jnp.dot(fv, w1_vmem[...], preferred_element_type=jnp.float32)
        h = jnp.maximum(h + b1_ref[...], 0.0)
        logits = jnp.dot(h, w2_ref[...], preferred_element_type=jnp.float32)
        logits = logits + b2_ref[...]
        m = jnp.max(logits, axis=-1, keepdims=True)
        e = jnp.exp(logits - m)
        w = e / jnp.sum(e, axis=-1, keepdims=True)
        rw_ref[...] = w
        ncols = w.shape[-1]
        ids = jax.lax.broadcasted_iota(jnp.int32, w.shape, 1)
        m1 = jnp.max(w, axis=-1, keepdims=True)
        i1 = jnp.min(jnp.where(w == m1, ids, ncols), axis=-1, keepdims=True)
        wm = jnp.where(ids == i1, -jnp.inf, w)
        m2 = jnp.max(wm, axis=-1, keepdims=True)
        i2 = jnp.min(jnp.where(wm == m2, ids, ncols), axis=-1, keepdims=True)
        col = jax.lax.broadcasted_iota(jnp.int32, idx_ref.shape, 1)
        idx_ref[...] = jnp.where(col == 0, i1, i2)


@jax.jit
def kernel(hidden_states, W1, b1, W2, b2):
    B, S, H = hidden_states.shape
    E = W2.shape[1]
    x2d = hidden_states.reshape(B * S, H)
    n_blk = (B * S) // _R_BLK
    n_steps = n_blk // _N_STREAMS
    blk_per_b = S // _R_BLK

    b1r = b1.reshape(1, H)
    b2r = b2.reshape(1, E)

    body = functools.partial(_router_body, n_steps=n_steps, n_blk=n_blk,
                             blk_per_b=blk_per_b, s_total=S)

    stream_specs = [
        pl.BlockSpec((_R_BLK, H), functools.partial(
            lambda s, i: (i + s * n_steps, 0), s))
        for s in range(_N_STREAMS)
    ]

    rw, idx = pl.pallas_call(
        body,
        grid=(n_steps,),
        in_specs=stream_specs + [
            pl.BlockSpec(memory_space=pl.ANY),
            pl.BlockSpec((1, H), lambda i: (0, 0)),
            pl.BlockSpec((H, E), lambda i: (0, 0)),
            pl.BlockSpec((1, E), lambda i: (0, 0)),
        ],
        out_specs=[
            pl.BlockSpec((B, E), lambda i: (0, 0)),
            pl.BlockSpec((B, 2), lambda i: (0, 0)),
        ],
        out_shape=[
            jax.ShapeDtypeStruct((B, E), jnp.float32),
            jax.ShapeDtypeStruct((B, 2), jnp.int32),
        ],
        scratch_shapes=[
            pltpu.VMEM((n_blk, H), jnp.float32),
            pltpu.VMEM((H, H), jnp.float32),
            pltpu.SemaphoreType.DMA,
        ],
    )(*([x2d] * _N_STREAMS), W1, b1r, W2, b2r)
    return rw, idx
